# Initial kernel scaffold; baseline (speedup 1.0000x reference)
import jax
import jax.numpy as jnp
from jax.experimental import pallas as pl


def kernel(x, table, W, b):
    raise NotImplementedError("write your pallas kernel here")


# speedup vs baseline: 3.6567x; 3.6567x over previous
import functools

import jax
import jax.numpy as jnp
from jax import lax
from jax.experimental import pallas as pl
from jax.experimental.pallas import tpu as pltpu
from jax.experimental.pallas import tpu_sc as plsc


def _proj_body(table_ref, w_ref, b_ref, out_ref):
    out_ref[...] = (
        jnp.dot(table_ref[...], w_ref[...], preferred_element_type=jnp.float32)
        + b_ref[...]
    )


@functools.lru_cache(maxsize=None)
def _make_project(V, D, blk):
    grid = V // blk
    return pl.pallas_call(
        _proj_body,
        grid=(grid,),
        in_specs=[
            pl.BlockSpec((blk, D), lambda i: (i, 0)),
            pl.BlockSpec((D, D), lambda i: (0, 0)),
            pl.BlockSpec((1, D), lambda i: (0, 0)),
        ],
        out_specs=pl.BlockSpec((blk, D), lambda i: (i, 0)),
        out_shape=jax.ShapeDtypeStruct((V, D), jnp.float32),
    )


@functools.lru_cache(maxsize=None)
def _make_gather(V, D, B):
    info = plsc.get_sparse_core_info()
    NC, NS = info.num_cores, info.num_subcores
    NW = NC * NS
    assert B % NW == 0
    b_per_w = B // NW
    CH = 1024
    assert b_per_w % CH == 0
    n_ch = b_per_w // CH
    mesh = plsc.VectorSubcoreMesh(core_axis_name="c", subcore_axis_name="s")

    @functools.partial(
        pl.kernel,
        mesh=mesh,
        compiler_params=pltpu.CompilerParams(use_tc_tiling_on_sc=False),
        out_type=jax.ShapeDtypeStruct((B, D), jnp.float32),
        scratch_types=[
            pltpu.VMEM((CH,), jnp.int32),
            pltpu.VMEM((CH, D), jnp.float32),
            pltpu.SemaphoreType.DMA,
        ],
    )
    def gather_kernel(table_hbm, idx_hbm, out_hbm, idx_v, rows_v, sem):
        wid = lax.axis_index("s") * NC + lax.axis_index("c")
        base = wid * b_per_w

        def body(i, carry):
            off = base + i * CH
            pltpu.sync_copy(idx_hbm.at[pl.ds(off, CH)], idx_v)
            pltpu.async_copy(table_hbm.at[idx_v], rows_v, sem).wait()
            pltpu.sync_copy(rows_v, out_hbm.at[pl.ds(off, CH)])
            return carry

        lax.fori_loop(0, n_ch, body, 0)

    return gather_kernel


def kernel(x, table, W, b):
    B, L = x.shape
    V, D = table.shape
    proj = _make_project(V, D, 2000)(table, W, b.reshape(1, D))
    idx = x.reshape(B * L).astype(jnp.int32)
    out = _make_gather(V, D, B * L)(proj, idx)
    return out.reshape(B, L, D)


# speedup vs baseline: 3.7511x; 1.0258x over previous
import functools

import jax
import jax.numpy as jnp
from jax import lax
from jax.experimental import pallas as pl
from jax.experimental.pallas import tpu as pltpu
from jax.experimental.pallas import tpu_sc as plsc


def _proj_body(table_ref, w_ref, b_ref, out_ref):
    out_ref[...] = (
        jnp.dot(table_ref[...], w_ref[...], preferred_element_type=jnp.float32)
        + b_ref[...]
    )


@functools.lru_cache(maxsize=None)
def _make_project(V, D, blk):
    grid = V // blk
    return pl.pallas_call(
        _proj_body,
        grid=(grid,),
        in_specs=[
            pl.BlockSpec((blk, D), lambda i: (i, 0)),
            pl.BlockSpec((D, D), lambda i: (0, 0)),
            pl.BlockSpec((1, D), lambda i: (0, 0)),
        ],
        out_specs=pl.BlockSpec((blk, D), lambda i: (i, 0)),
        out_shape=jax.ShapeDtypeStruct((V, D), jnp.float32),
    )


@functools.lru_cache(maxsize=None)
def _make_gather(V, D, B):
    info = plsc.get_sparse_core_info()
    NC, NS = info.num_cores, info.num_subcores
    NW = NC * NS
    assert B % NW == 0
    b_per_w = B // NW
    NBUF = 4
    CH = 320
    assert b_per_w % (CH * NBUF) == 0
    n_ch = b_per_w // CH
    mesh = plsc.VectorSubcoreMesh(core_axis_name="c", subcore_axis_name="s")

    @functools.partial(
        pl.kernel,
        mesh=mesh,
        compiler_params=pltpu.CompilerParams(use_tc_tiling_on_sc=False),
        out_type=jax.ShapeDtypeStruct((B, D), jnp.float32),
        scratch_types=[
            pltpu.VMEM((b_per_w,), jnp.int32),
            pltpu.VMEM((NBUF, CH, D), jnp.float32),
            pltpu.SemaphoreType.DMA,
            pltpu.SemaphoreType.DMA,
            pltpu.SemaphoreType.DMA,
            pltpu.SemaphoreType.DMA,
            pltpu.SemaphoreType.DMA,
        ],
    )
    def gather_kernel(table_hbm, idx_hbm, out_hbm, idx_v, rows_v, gsem,
                      w0, w1, w2, w3):
        wsem = (w0, w1, w2, w3)
        wid = lax.axis_index("s") * NC + lax.axis_index("c")
        base = wid * b_per_w

        def fire_gather(c, b):
            pltpu.async_copy(
                table_hbm.at[idx_v.at[pl.ds(c * CH, CH)]],
                rows_v.at[b], gsem)

        def wait_gather(b):
            pltpu.make_async_copy(
                table_hbm.at[pl.ds(0, CH)], rows_v.at[b], gsem).wait()

        def wait_wb(b):
            pltpu.make_async_copy(
                rows_v.at[b], out_hbm.at[pl.ds(0, CH)], wsem[b]).wait()

        pltpu.sync_copy(idx_hbm.at[pl.ds(base, b_per_w)], idx_v)
        for b in range(NBUF):
            fire_gather(b, b)

        def body(i, carry):
            for b in range(NBUF):
                c = i * NBUF + b
                wait_gather(b)
                pltpu.async_copy(
                    rows_v.at[b], out_hbm.at[pl.ds(base + c * CH, CH)],
                    wsem[b])
                nxt = c + NBUF

                @pl.when(nxt < n_ch)
                def _():
                    wait_wb(b)
                    fire_gather(nxt, b)

            return carry

        lax.fori_loop(0, n_ch // NBUF, body, 0)
        for b in range(NBUF):
            wait_wb(b)

    return gather_kernel


def kernel(x, table, W, b):
    B, L = x.shape
    V, D = table.shape
    proj = _make_project(V, D, 2000)(table, W, b.reshape(1, D))
    idx = x.reshape(B * L).astype(jnp.int32)
    out = _make_gather(V, D, B * L)(proj, idx)
    return out.reshape(B, L, D)


# speedup vs baseline: 3.7660x; 1.0040x over previous
import functools

import jax
import jax.numpy as jnp
from jax import lax
from jax.experimental import pallas as pl
from jax.experimental.pallas import tpu as pltpu
from jax.experimental.pallas import tpu_sc as plsc


def _proj_body(table_ref, w_ref, b_ref, out_ref):
    out_ref[...] = (
        jnp.dot(table_ref[...], w_ref[...], preferred_element_type=jnp.float32)
        + b_ref[...]
    )


@functools.lru_cache(maxsize=None)
def _make_project(V, D, blk):
    grid = V // blk
    return pl.pallas_call(
        _proj_body,
        grid=(grid,),
        in_specs=[
            pl.BlockSpec((blk, D), lambda i: (i, 0)),
            pl.BlockSpec((D, D), lambda i: (0, 0)),
            pl.BlockSpec((1, D), lambda i: (0, 0)),
        ],
        out_specs=pl.BlockSpec((blk, D), lambda i: (i, 0)),
        out_shape=jax.ShapeDtypeStruct((V, D), jnp.float32),
    )


@functools.lru_cache(maxsize=None)
def _make_gather(V, D, B, L):
    info = plsc.get_sparse_core_info()
    NC, NS = info.num_cores, info.num_subcores
    NW = NC * NS
    assert B % NW == 0
    r_per_w = B // NW
    NBUF = 4
    assert r_per_w % NBUF == 0
    mesh = plsc.VectorSubcoreMesh(core_axis_name="c", subcore_axis_name="s")

    @functools.partial(
        pl.kernel,
        mesh=mesh,
        compiler_params=pltpu.CompilerParams(use_tc_tiling_on_sc=False),
        out_type=jax.ShapeDtypeStruct((B, L, D), jnp.float32),
        scratch_types=[
            pltpu.VMEM((r_per_w, L), jnp.int32),
            pltpu.VMEM((NBUF, L, D), jnp.float32),
            pltpu.SemaphoreType.DMA,
            pltpu.SemaphoreType.DMA,
            pltpu.SemaphoreType.DMA,
            pltpu.SemaphoreType.DMA,
            pltpu.SemaphoreType.DMA,
        ],
    )
    def gather_kernel(table_hbm, idx_hbm, out_hbm, idx_v, rows_v, gsem,
                      w0, w1, w2, w3):
        wsem = (w0, w1, w2, w3)
        wid = lax.axis_index("s") * NC + lax.axis_index("c")
        base = wid * r_per_w

        def fire_gather(r, b):
            pltpu.async_copy(
                table_hbm.at[idx_v.at[r]], rows_v.at[b], gsem)

        def wait_gather(b):
            pltpu.make_async_copy(
                table_hbm.at[pl.ds(0, L)], rows_v.at[b], gsem).wait()

        def wait_wb(b):
            pltpu.make_async_copy(
                rows_v.at[b], out_hbm.at[0], wsem[b]).wait()

        pltpu.sync_copy(idx_hbm.at[pl.ds(base, r_per_w), :], idx_v)
        for b in range(NBUF):
            fire_gather(b, b)

        def body(i, carry):
            for b in range(NBUF):
                r = i * NBUF + b
                wait_gather(b)
                pltpu.async_copy(rows_v.at[b], out_hbm.at[base + r], wsem[b])
                nxt = r + NBUF

                @pl.when(nxt < r_per_w)
                def _():
                    wait_wb(b)
                    fire_gather(nxt, b)

            return carry

        lax.fori_loop(0, r_per_w // NBUF, body, 0)
        for b in range(NBUF):
            wait_wb(b)

    return gather_kernel


def kernel(x, table, W, b):
    B, L = x.shape
    V, D = table.shape
    proj = _make_project(V, D, 2000)(table, W, b.reshape(1, D))
    return _make_gather(V, D, B, L)(proj, x.astype(jnp.int32))
